# baseline (speedup 1.0000x reference)
import functools

import jax
import jax.numpy as jnp
from jax import lax
from jax.experimental import pallas as pl
from jax.experimental.pallas import tpu as pltpu
from jax.experimental.pallas import tpu_sc as plsc

E = 128
S = 200
B = 4096
C0 = 104
C1 = S - C0
L = 16
NC = 2
NS = 16
NW = NC * NS
BPW = B // NW
NFULL = S // L
TAIL = S - NFULL * L
NBUF = 3


def _tile_body(tok_hbm, table_hbm, out_hbm, tokt_v, idx_v, rows0, rows1,
               rows2, gsem0, gsem1, gsem2, osem0, osem1, osem2):
    wid = lax.axis_index("s") * NC + lax.axis_index("c")
    base = wid * BPW

    lane = lax.iota(jnp.int32, L)
    tail_mask = lane < TAIL
    tail_rows = jnp.minimum(lane + NFULL * L, S - 1)

    pltpu.sync_copy(tok_hbm.at[:, pl.ds(base, BPW)], tokt_v)

    @plsc.parallel_loop(0, BPW, unroll=2)
    def _r_loop(r):
        col = jnp.broadcast_to(r, (L,))
        for k in range(NFULL):
            v = plsc.load_gather(tokt_v, [lane + k * L, col])
            idx_v[r, pl.ds(k * L, L)] = v
        v = plsc.load_gather(tokt_v, [tail_rows, col])
        plsc.store_scatter(idx_v, [col, lane + NFULL * L], v, mask=tail_mask)

    bufs = (rows0, rows1, rows2)
    gsems = (gsem0, gsem1, gsem2)
    osems = (osem0, osem1, osem2)

    def gather(b, j):
        return tuple(
            pltpu.make_async_copy(
                table_hbm.at[idx_v.at[b, pl.ds(off, ln)]],
                bufs[j].at[pl.ds(off, ln)], gsems[j])
            for off, ln in ((0, 56), (56, 56), (112, 56), (168, 32)))

    def outcopy(b, j):
        return pltpu.make_async_copy(bufs[j], out_hbm.at[base + b], osems[j])

    def slot(bb, h):
        for c in gather(bb, h):
            c.wait()
        outcopy(bb, h).start()
        j2 = (h + 2) % NBUF

        @pl.when(bb + 2 < BPW)
        def _():
            @pl.when(bb >= 1)
            def _():
                outcopy(bb - 1, j2).wait()

            for c in gather(bb + 2, j2):
                c.start()

    for c in gather(0, 0) + gather(1, 1):
        c.start()

    @pl.loop(0, BPW - 2, step=NBUF)
    def _b_loop(b):
        for h in range(NBUF):
            slot(b + h, h)

    slot(BPW - 2, 0)
    slot(BPW - 1, 1)

    outcopy(BPW - 3, 2).wait()
    outcopy(BPW - 2, 0).wait()
    outcopy(BPW - 1, 1).wait()


@functools.partial(
    pl.kernel,
    out_type=jax.ShapeDtypeStruct((B, S, E), jnp.float32),
    mesh=plsc.VectorSubcoreMesh(core_axis_name="c", subcore_axis_name="s"),
    compiler_params=pltpu.CompilerParams(use_tc_tiling_on_sc=False,
                                         needs_layout_passes=False),
    scratch_types=[
        pltpu.VMEM((S, BPW), jnp.int32),
        pltpu.VMEM((BPW, S), jnp.int32),
        pltpu.VMEM((S, E), jnp.float32),
        pltpu.VMEM((S, E), jnp.float32),
        pltpu.VMEM((S, E), jnp.float32),
        pltpu.SemaphoreType.DMA,
        pltpu.SemaphoreType.DMA,
        pltpu.SemaphoreType.DMA,
        pltpu.SemaphoreType.DMA,
        pltpu.SemaphoreType.DMA,
        pltpu.SemaphoreType.DMA,
    ],
)
def _bow_encode(tok_hbm, table_hbm, out_hbm, tokt_v, idx_v, rows0, rows1,
                rows2, gsem0, gsem1, gsem2, osem0, osem1, osem2):
    _tile_body(tok_hbm, table_hbm, out_hbm, tokt_v, idx_v, rows0, rows1,
               rows2, gsem0, gsem1, gsem2, osem0, osem1, osem2)


def kernel(tokens, table):
    out = _bow_encode(tokens.astype(jnp.int32), table)
    return jnp.swapaxes(out, 1, 2)


# speedup vs baseline: 1.0013x; 1.0013x over previous
import functools

import jax
import jax.numpy as jnp
from jax import lax
from jax.experimental import pallas as pl
from jax.experimental.pallas import tpu as pltpu
from jax.experimental.pallas import tpu_sc as plsc

E = 128
S = 200
B = 4096
C0 = 104
C1 = S - C0
L = 16
NC = 2
NS = 16
NW = NC * NS
BPW = B // NW
NFULL = S // L
TAIL = S - NFULL * L
NBUF = 3


def _tile_body(tok_hbm, table_hbm, out_hbm, tokt_v, idx_v, rows0, rows1,
               rows2, gsem0, gsem1, gsem2, osem0, osem1, osem2):
    wid = lax.axis_index("s") * NC + lax.axis_index("c")
    base = wid * BPW

    lane = lax.iota(jnp.int32, L)
    tail_mask = lane < TAIL
    tail_rows = jnp.minimum(lane + NFULL * L, S - 1)

    pltpu.sync_copy(tok_hbm.at[:, pl.ds(base, BPW)], tokt_v)

    @plsc.parallel_loop(0, BPW, unroll=2)
    def _r_loop(r):
        col = jnp.broadcast_to(r, (L,))
        for k in range(NFULL):
            v = plsc.load_gather(tokt_v, [lane + k * L, col])
            idx_v[r, pl.ds(k * L, L)] = v
        v = plsc.load_gather(tokt_v, [tail_rows, col])
        plsc.store_scatter(idx_v, [col, lane + NFULL * L], v, mask=tail_mask)

    bufs = (rows0, rows1, rows2)
    gsems = (gsem0, gsem1, gsem2)
    osems = (osem0, osem1, osem2)

    def gather(b, j):
        return (
            pltpu.make_async_copy(
                table_hbm.at[idx_v.at[b, pl.ds(0, C0)]],
                bufs[j].at[pl.ds(0, C0)], gsems[j]),
            pltpu.make_async_copy(
                table_hbm.at[idx_v.at[b, pl.ds(C0, C1)]],
                bufs[j].at[pl.ds(C0, C1)], gsems[j]),
        )

    def outcopy(b, j):
        return pltpu.make_async_copy(bufs[j], out_hbm.at[base + b], osems[j])

    def slot(bb, h):
        for c in gather(bb, h):
            c.wait()
        outcopy(bb, h).start()
        j2 = (h + 2) % NBUF

        @pl.when(bb + 2 < BPW)
        def _():
            @pl.when(bb >= 1)
            def _():
                outcopy(bb - 1, j2).wait()

            for c in gather(bb + 2, j2):
                c.start()

    for c in gather(0, 0) + gather(1, 1):
        c.start()

    @pl.loop(0, BPW - 2, step=NBUF)
    def _b_loop(b):
        for h in range(NBUF):
            slot(b + h, h)

    slot(BPW - 2, 0)
    slot(BPW - 1, 1)

    outcopy(BPW - 3, 2).wait()
    outcopy(BPW - 2, 0).wait()
    outcopy(BPW - 1, 1).wait()


@functools.partial(
    pl.kernel,
    out_type=jax.ShapeDtypeStruct((B, S, E), jnp.float32),
    mesh=plsc.VectorSubcoreMesh(core_axis_name="c", subcore_axis_name="s"),
    compiler_params=pltpu.CompilerParams(use_tc_tiling_on_sc=False,
                                         needs_layout_passes=False),
    scratch_types=[
        pltpu.VMEM((S, BPW), jnp.int32),
        pltpu.VMEM((BPW, S), jnp.int32),
        pltpu.VMEM((S, E), jnp.float32),
        pltpu.VMEM((S, E), jnp.float32),
        pltpu.VMEM((S, E), jnp.float32),
        pltpu.SemaphoreType.DMA,
        pltpu.SemaphoreType.DMA,
        pltpu.SemaphoreType.DMA,
        pltpu.SemaphoreType.DMA,
        pltpu.SemaphoreType.DMA,
        pltpu.SemaphoreType.DMA,
    ],
)
def _bow_encode(tok_hbm, table_hbm, out_hbm, tokt_v, idx_v, rows0, rows1,
                rows2, gsem0, gsem1, gsem2, osem0, osem1, osem2):
    _tile_body(tok_hbm, table_hbm, out_hbm, tokt_v, idx_v, rows0, rows1,
               rows2, gsem0, gsem1, gsem2, osem0, osem1, osem2)


def kernel(tokens, table):
    out = _bow_encode(tokens.astype(jnp.int32), table)
    return jnp.swapaxes(out, 1, 2)
